# baseline (speedup 1.0000x reference)
import jax
import jax.numpy as jnp
from jax.experimental import pallas as pl
from jax.experimental.pallas import tpu as pltpu

TOPK = 4
LTILE = 512
LSEQ = 4096


def _body(text_tile_ref, text_hbm, vis_hbm, w_ref, b_ref,
          out_t_hbm, out_v_hbm,
          scores_ref, zeros_ref, small_t_ref, small_v_ref,
          tk_ref, vk_ref, sem_out, sem_g):
    i = pl.program_id(0)
    nsteps = pl.num_programs(0)
    B, _, D = text_tile_ref.shape
    L = scores_ref.shape[1]

    @pl.when(i == 0)
    def _():
        zeros_ref[...] = jnp.zeros_like(zeros_ref)
        for bb in range(B):
            pltpu.make_async_copy(
                vis_hbm.at[bb].at[pl.ds(0, TOPK), :], vk_ref.at[bb],
                sem_g.at[B * TOPK + bb]).start()

    x = text_tile_ref[...]
    scores_ref[:, pl.ds(i * LTILE, LTILE)] = jnp.sum(x * x, axis=-1)

    @pl.when(i == nsteps - 1)
    def _():
        sc = scores_ref[...]
        lane_idx = jax.lax.broadcasted_iota(jnp.int32, (B, L), 1)
        big = jnp.int32(2**30)
        gathers = []
        for bb in range(B):
            row = sc[bb:bb + 1, :]
            li = lane_idx[bb:bb + 1, :]
            for t in range(TOPK):
                m = jnp.max(row)
                a = jnp.min(jnp.where(row == m, li, big))
                cp = pltpu.make_async_copy(
                    text_hbm.at[bb].at[pl.ds(a, 1), :],
                    tk_ref.at[bb].at[pl.ds(t, 1), :],
                    sem_g.at[bb * TOPK + t])
                cp.start()
                gathers.append(cp)
                row = jnp.where(li == a, jnp.float32(-1.0), row)
        for bb in range(B):
            gathers.append(pltpu.make_async_copy(
                vis_hbm.at[bb].at[pl.ds(0, TOPK), :], vk_ref.at[bb],
                sem_g.at[B * TOPK + bb]))
        for cp in gathers:
            cp.wait()

        def _dot(lhs, rhs, dims):
            return jax.lax.dot_general(
                lhs.astype(jnp.bfloat16), rhs.astype(jnp.bfloat16), dims,
                preferred_element_type=jnp.float32)

        w = w_ref[...]
        bias = b_ref[...]
        small_t_ref[...] = jnp.zeros_like(small_t_ref)
        small_v_ref[...] = jnp.zeros_like(small_v_ref)
        for bb in range(B):
            tk = tk_ref[bb]
            vk = vk_ref[bb]
            proj = _dot(tk, w, (((1,), (1,)), ((), ()))) + bias
            sim = _dot(proj, vk, (((1,), (1,)), ((), ())))
            edges = jax.nn.softmax(sim, axis=-1)
            small_t_ref[bb, 0:TOPK, :] = _dot(
                edges, vk, (((1,), (0,)), ((), ())))
            small_v_ref[bb, 0:TOPK, :] = _dot(
                edges, tk, (((0,), (0,)), ((), ())))

        copies = [
            pltpu.make_async_copy(
                small_t_ref, out_t_hbm.at[:, pl.ds(0, 8), :],
                sem_out.at[0]),
            pltpu.make_async_copy(
                small_v_ref, out_v_hbm.at[:, pl.ds(0, 8), :],
                sem_out.at[1]),
            pltpu.make_async_copy(
                zeros_ref.at[:, pl.ds(0, LTILE - 8), :],
                out_t_hbm.at[:, pl.ds(8, LTILE - 8), :], sem_out.at[2]),
            pltpu.make_async_copy(
                zeros_ref.at[:, pl.ds(0, LTILE - 8), :],
                out_v_hbm.at[:, pl.ds(8, LTILE - 8), :], sem_out.at[3]),
        ]
        n = 4
        for j in range(1, nsteps):
            copies.append(pltpu.make_async_copy(
                zeros_ref, out_t_hbm.at[:, pl.ds(j * LTILE, LTILE), :],
                sem_out.at[n])); n += 1
            copies.append(pltpu.make_async_copy(
                zeros_ref, out_v_hbm.at[:, pl.ds(j * LTILE, LTILE), :],
                sem_out.at[n])); n += 1
        for cp in copies:
            cp.start()
        for cp in copies:
            cp.wait()


@jax.jit
def kernel(text_feats, visual_feats, W, b):
    B, L, D = text_feats.shape
    nsteps = L // LTILE

    out_t, out_v = pl.pallas_call(
        _body,
        grid=(nsteps,),
        in_specs=[
            pl.BlockSpec((B, LTILE, D), lambda i: (0, i, 0)),
            pl.BlockSpec(memory_space=pl.ANY),
            pl.BlockSpec(memory_space=pl.ANY),
            pl.BlockSpec((D, D), lambda i: (0, 0)),
            pl.BlockSpec((1, D), lambda i: (0, 0)),
        ],
        out_specs=[
            pl.BlockSpec(memory_space=pl.ANY),
            pl.BlockSpec(memory_space=pl.ANY),
        ],
        out_shape=[
            jax.ShapeDtypeStruct((B, L, D), jnp.float32),
            jax.ShapeDtypeStruct((B, L, D), jnp.float32),
        ],
        scratch_shapes=[
            pltpu.VMEM((B, L), jnp.float32),
            pltpu.VMEM((B, LTILE, D), jnp.float32),
            pltpu.VMEM((B, 8, D), jnp.float32),
            pltpu.VMEM((B, 8, D), jnp.float32),
            pltpu.VMEM((B, TOPK, D), jnp.float32),
            pltpu.VMEM((B, TOPK, D), jnp.float32),
            pltpu.SemaphoreType.DMA((2 * (L // LTILE) + 2,)),
            pltpu.SemaphoreType.DMA((B * TOPK + B,)),
        ],
    )(text_feats, text_feats, visual_feats, W, b.reshape(1, D))
    return (out_t, out_v)


# speedup vs baseline: 1.2511x; 1.2511x over previous
import jax
import jax.numpy as jnp
from jax.experimental import pallas as pl
from jax.experimental.pallas import tpu as pltpu

TOPK = 4
LTILE = 512


def _body(text_tile_ref, text_hbm, vis_hbm, w_ref, b_ref,
          out_t_hbm, out_v_hbm,
          scores_ref, zeros_ref, small_t_ref, small_v_ref,
          tk_ref, vk_ref, sem_out, sem_g):
    i = pl.program_id(0)
    nsteps = pl.num_programs(0)
    B, _, D = text_tile_ref.shape
    L = scores_ref.shape[1]

    @pl.when(i == 0)
    def _():
        zeros_ref[...] = jnp.zeros_like(zeros_ref)
        for j in range(1, nsteps):
            pltpu.make_async_copy(
                zeros_ref, out_t_hbm.at[:, pl.ds(j * LTILE, LTILE), :],
                sem_out.at[2 * j]).start()
            pltpu.make_async_copy(
                zeros_ref, out_v_hbm.at[:, pl.ds(j * LTILE, LTILE), :],
                sem_out.at[2 * j + 1]).start()
        for bb in range(B):
            pltpu.make_async_copy(
                vis_hbm.at[bb].at[pl.ds(0, TOPK), :], vk_ref.at[bb],
                sem_g.at[B * TOPK + bb]).start()

    x = text_tile_ref[...]
    scores_ref[:, pl.ds(i * LTILE, LTILE)] = jnp.sum(x * x, axis=-1)

    @pl.when(i == nsteps - 1)
    def _():
        sc = scores_ref[...]
        lane_idx = jax.lax.broadcasted_iota(jnp.int32, (B, L), 1)
        big = jnp.int32(2**30)
        gathers = []
        for bb in range(B):
            row = sc[bb:bb + 1, :]
            li = lane_idx[bb:bb + 1, :]
            for t in range(TOPK):
                m = jnp.max(row)
                a = jnp.min(jnp.where(row == m, li, big))
                cp = pltpu.make_async_copy(
                    text_hbm.at[bb].at[pl.ds(a, 1), :],
                    tk_ref.at[bb].at[pl.ds(t, 1), :],
                    sem_g.at[bb * TOPK + t])
                cp.start()
                gathers.append(cp)
                row = jnp.where(li == a, jnp.float32(-1.0), row)
        for bb in range(B):
            gathers.append(pltpu.make_async_copy(
                vis_hbm.at[bb].at[pl.ds(0, TOPK), :], vk_ref.at[bb],
                sem_g.at[B * TOPK + bb]))
        for cp in gathers:
            cp.wait()

        def _dot(lhs, rhs, dims):
            return jax.lax.dot_general(
                lhs.astype(jnp.bfloat16), rhs.astype(jnp.bfloat16), dims,
                preferred_element_type=jnp.float32)

        w = w_ref[...]
        bias = b_ref[...]
        small_t_ref[...] = jnp.zeros_like(small_t_ref)
        small_v_ref[...] = jnp.zeros_like(small_v_ref)
        for bb in range(B):
            tk = tk_ref[bb]
            vk = vk_ref[bb]
            proj = _dot(tk, w, (((1,), (1,)), ((), ()))) + bias
            sim = _dot(proj, vk, (((1,), (1,)), ((), ())))
            edges = jax.nn.softmax(sim, axis=-1)
            small_t_ref[bb, 0:TOPK, :] = _dot(
                edges, vk, (((1,), (0,)), ((), ())))
            small_v_ref[bb, 0:TOPK, :] = _dot(
                edges, tk, (((0,), (0,)), ((), ())))

        finals = [
            pltpu.make_async_copy(
                small_t_ref, out_t_hbm.at[:, pl.ds(0, 8), :],
                sem_out.at[0]),
            pltpu.make_async_copy(
                small_v_ref, out_v_hbm.at[:, pl.ds(0, 8), :],
                sem_out.at[1]),
            pltpu.make_async_copy(
                zeros_ref.at[:, pl.ds(0, LTILE - 8), :],
                out_t_hbm.at[:, pl.ds(8, LTILE - 8), :],
                sem_out.at[2 * nsteps]),
            pltpu.make_async_copy(
                zeros_ref.at[:, pl.ds(0, LTILE - 8), :],
                out_v_hbm.at[:, pl.ds(8, LTILE - 8), :],
                sem_out.at[2 * nsteps + 1]),
        ]
        for cp in finals:
            cp.start()
        for j in range(1, nsteps):
            pltpu.make_async_copy(
                zeros_ref, out_t_hbm.at[:, pl.ds(j * LTILE, LTILE), :],
                sem_out.at[2 * j]).wait()
            pltpu.make_async_copy(
                zeros_ref, out_v_hbm.at[:, pl.ds(j * LTILE, LTILE), :],
                sem_out.at[2 * j + 1]).wait()
        for cp in finals:
            cp.wait()


@jax.jit
def kernel(text_feats, visual_feats, W, b):
    B, L, D = text_feats.shape
    nsteps = L // LTILE

    out_t, out_v = pl.pallas_call(
        _body,
        grid=(nsteps,),
        in_specs=[
            pl.BlockSpec((B, LTILE, D), lambda i: (0, i, 0)),
            pl.BlockSpec(memory_space=pl.ANY),
            pl.BlockSpec(memory_space=pl.ANY),
            pl.BlockSpec((D, D), lambda i: (0, 0)),
            pl.BlockSpec((1, D), lambda i: (0, 0)),
        ],
        out_specs=[
            pl.BlockSpec(memory_space=pl.ANY),
            pl.BlockSpec(memory_space=pl.ANY),
        ],
        out_shape=[
            jax.ShapeDtypeStruct((B, L, D), jnp.float32),
            jax.ShapeDtypeStruct((B, L, D), jnp.float32),
        ],
        scratch_shapes=[
            pltpu.VMEM((B, L), jnp.float32),
            pltpu.VMEM((B, LTILE, D), jnp.float32),
            pltpu.VMEM((B, 8, D), jnp.float32),
            pltpu.VMEM((B, 8, D), jnp.float32),
            pltpu.VMEM((B, TOPK, D), jnp.float32),
            pltpu.VMEM((B, TOPK, D), jnp.float32),
            pltpu.SemaphoreType.DMA((2 * nsteps + 2,)),
            pltpu.SemaphoreType.DMA((B * TOPK + B,)),
        ],
    )(text_feats, text_feats, visual_feats, W, b.reshape(1, D))
    return (out_t, out_v)
